# Initial kernel scaffold; baseline (speedup 1.0000x reference)
import jax
import jax.numpy as jnp
from jax.experimental import pallas as pl


def kernel(queries, keys, m_persistent):
    raise NotImplementedError("write your pallas kernel here")


# speedup vs baseline: 2.8166x; 2.8166x over previous
import jax
import jax.numpy as jnp
from jax.experimental import pallas as pl
from jax.experimental.pallas import tpu as pltpu

_T = 8192
_DIM = 1024
_CHUNK = 128
_DIMH = _DIM // 2
_NORM_PERSISTENT = 1024.0


def _qkproj_kernel(q_ref, kf_ref, kh_ref, mp_ref, out_ref, m_acc, n_acc):
    i = pl.program_id(1)

    @pl.when(i == 0)
    def _init():
        m_acc[...] = mp_ref[...]
        n_acc[0, 0] = _NORM_PERSISTENT

    q = q_ref[...]
    kf = kf_ref[...]
    kh = kh_ref[...]

    row = jax.lax.broadcasted_iota(jnp.int32, (_CHUNK, _CHUNK), 0)
    col = jax.lax.broadcasted_iota(jnp.int32, (_CHUNK, _CHUNK), 1)
    causal = (col <= row)

    ss = jnp.sum(kf * kf, axis=1, keepdims=True)
    csum = jnp.dot(causal.astype(jnp.float32), ss,
                   preferred_element_type=jnp.float32)
    norms = n_acc[0, 0] + csum
    n_acc[0, 0] = n_acc[0, 0] + jnp.sum(ss)

    scores = jax.lax.dot_general(q, kf, (((1,), (1,)), ((), ())),
                                 preferred_element_type=jnp.float32)
    scores = jnp.where(causal, scores, 0.0)

    m = m_acc[...]
    out = jax.lax.dot_general(q, m, (((1,), (1,)), ((), ())),
                              preferred_element_type=jnp.float32)
    out = out + jax.lax.dot_general(scores, kh, (((1,), (0,)), ((), ())),
                                    preferred_element_type=jnp.float32)
    out_ref[...] = out / jnp.maximum(norms, 1e-8)

    m_acc[...] = m + jax.lax.dot_general(kh, kf, (((0,), (0,)), ((), ())),
                                         preferred_element_type=jnp.float32)


def kernel(queries, keys, m_persistent):
    t_len, dim = queries.shape
    n_chunks = t_len // _CHUNK
    return pl.pallas_call(
        _qkproj_kernel,
        out_shape=jax.ShapeDtypeStruct((t_len, dim), jnp.float32),
        grid=(2, n_chunks),
        in_specs=[
            pl.BlockSpec((_CHUNK, dim), lambda j, i: (i, 0)),
            pl.BlockSpec((_CHUNK, dim), lambda j, i: (i, 0)),
            pl.BlockSpec((_CHUNK, _DIMH), lambda j, i: (i, j)),
            pl.BlockSpec((_DIMH, dim), lambda j, i: (j, 0)),
        ],
        out_specs=pl.BlockSpec((_CHUNK, _DIMH), lambda j, i: (i, j)),
        scratch_shapes=[
            pltpu.VMEM((_DIMH, dim), jnp.float32),
            pltpu.SMEM((1, 1), jnp.float32),
        ],
        compiler_params=pltpu.CompilerParams(
            dimension_semantics=("parallel", "arbitrary"),
        ),
        name="qkprojection",
    )(queries, keys, keys, m_persistent)


# speedup vs baseline: 2.8278x; 1.0040x over previous
import jax
import jax.numpy as jnp
from jax.experimental import pallas as pl
from jax.experimental.pallas import tpu as pltpu

_T = 8192
_DIM = 1024
_CHUNK = 128
_DIMH = _DIM // 2
_NORM_PERSISTENT = 1024.0


def _qkproj_kernel(q_ref, kf_ref, kh_ref, mp_ref, out_ref, m_acc, n_acc):
    i = pl.program_id(1)

    @pl.when(i == 0)
    def _init():
        m_acc[...] = mp_ref[...]
        n_acc[0, 0] = _NORM_PERSISTENT

    q = q_ref[...]
    kf = kf_ref[...]
    kh = kh_ref[...]
    qb = q.astype(jnp.bfloat16)
    kfb = kf.astype(jnp.bfloat16)
    khb = kh.astype(jnp.bfloat16)

    row = jax.lax.broadcasted_iota(jnp.int32, (_CHUNK, _CHUNK), 0)
    col = jax.lax.broadcasted_iota(jnp.int32, (_CHUNK, _CHUNK), 1)
    causal = (col <= row)

    ss = jnp.sum(kf * kf, axis=1, keepdims=True)
    csum = jnp.dot(causal.astype(jnp.float32), ss,
                   preferred_element_type=jnp.float32)
    norms = n_acc[0, 0] + csum
    n_acc[0, 0] = n_acc[0, 0] + jnp.sum(ss)

    scores = jax.lax.dot_general(qb, kfb, (((1,), (1,)), ((), ())),
                                 preferred_element_type=jnp.float32)
    scores = jnp.where(causal, scores, 0.0).astype(jnp.bfloat16)

    m = m_acc[...]
    out = jax.lax.dot_general(qb, m.astype(jnp.bfloat16),
                              (((1,), (1,)), ((), ())),
                              preferred_element_type=jnp.float32)
    out = out + jax.lax.dot_general(scores, khb, (((1,), (0,)), ((), ())),
                                    preferred_element_type=jnp.float32)
    out_ref[...] = out / jnp.maximum(norms, 1e-8)

    m_acc[...] = m + jax.lax.dot_general(khb, kfb, (((0,), (0,)), ((), ())),
                                         preferred_element_type=jnp.float32)


def kernel(queries, keys, m_persistent):
    t_len, dim = queries.shape
    n_chunks = t_len // _CHUNK
    return pl.pallas_call(
        _qkproj_kernel,
        out_shape=jax.ShapeDtypeStruct((t_len, dim), jnp.float32),
        grid=(2, n_chunks),
        in_specs=[
            pl.BlockSpec((_CHUNK, dim), lambda j, i: (i, 0)),
            pl.BlockSpec((_CHUNK, dim), lambda j, i: (i, 0)),
            pl.BlockSpec((_CHUNK, _DIMH), lambda j, i: (i, j)),
            pl.BlockSpec((_DIMH, dim), lambda j, i: (j, 0)),
        ],
        out_specs=pl.BlockSpec((_CHUNK, _DIMH), lambda j, i: (i, j)),
        scratch_shapes=[
            pltpu.VMEM((_DIMH, dim), jnp.float32),
            pltpu.SMEM((1, 1), jnp.float32),
        ],
        compiler_params=pltpu.CompilerParams(
            dimension_semantics=("parallel", "arbitrary"),
        ),
        name="qkprojection",
    )(queries, keys, keys, m_persistent)


# speedup vs baseline: 6.8080x; 2.4075x over previous
import jax
import jax.numpy as jnp
from jax.experimental import pallas as pl
from jax.experimental.pallas import tpu as pltpu

_CHUNK = 256
_NORM_PERSISTENT = 1024.0


def _qkproj_kernel(q_ref, k_ref, mp_ref, out_ref, m_acc, n_acc):
    i = pl.program_id(0)

    @pl.when(i == 0)
    def _init():
        m_acc[...] = mp_ref[...]
        n_acc[0, 0] = _NORM_PERSISTENT

    q = q_ref[...]
    k = k_ref[...]

    row = jax.lax.broadcasted_iota(jnp.int32, (_CHUNK, _CHUNK), 0)
    col = jax.lax.broadcasted_iota(jnp.int32, (_CHUNK, _CHUNK), 1)
    causal = (col <= row)

    ss = jnp.sum(k * k, axis=1, keepdims=True)
    csum = jnp.dot(causal.astype(jnp.float32), ss,
                   preferred_element_type=jnp.float32)
    norms = n_acc[0, 0] + csum
    n_acc[0, 0] = n_acc[0, 0] + jnp.sum(ss)

    scores = jax.lax.dot_general(q, k, (((1,), (1,)), ((), ())),
                                 preferred_element_type=jnp.float32)
    scores = jnp.where(causal, scores, 0.0)

    m = m_acc[...]
    out = jax.lax.dot_general(q, m, (((1,), (1,)), ((), ())),
                              preferred_element_type=jnp.float32)
    out = out + jax.lax.dot_general(scores, k, (((1,), (0,)), ((), ())),
                                    preferred_element_type=jnp.float32)
    out_ref[...] = out / jnp.maximum(norms, 1e-8)

    m_acc[...] = m + jax.lax.dot_general(k, k, (((0,), (0,)), ((), ())),
                                         preferred_element_type=jnp.float32)


def kernel(queries, keys, m_persistent):
    t_len, dim = queries.shape
    n_chunks = t_len // _CHUNK
    return pl.pallas_call(
        _qkproj_kernel,
        out_shape=jax.ShapeDtypeStruct((t_len, dim), jnp.float32),
        grid=(n_chunks,),
        in_specs=[
            pl.BlockSpec((_CHUNK, dim), lambda i: (i, 0)),
            pl.BlockSpec((_CHUNK, dim), lambda i: (i, 0)),
            pl.BlockSpec((dim, dim), lambda i: (0, 0)),
        ],
        out_specs=pl.BlockSpec((_CHUNK, dim), lambda i: (i, 0)),
        scratch_shapes=[
            pltpu.VMEM((dim, dim), jnp.float32),
            pltpu.SMEM((1, 1), jnp.float32),
        ],
        compiler_params=pltpu.CompilerParams(
            dimension_semantics=("arbitrary",),
        ),
        name="qkprojection",
    )(queries, keys, m_persistent)


# speedup vs baseline: 7.1956x; 1.0569x over previous
import jax
import jax.numpy as jnp
from jax.experimental import pallas as pl
from jax.experimental.pallas import tpu as pltpu

_CHUNK = 512
_NORM_PERSISTENT = 1024.0


def _qkproj_kernel(q_ref, k_ref, mp_ref, out_ref, m_acc, n_acc):
    i = pl.program_id(0)

    @pl.when(i == 0)
    def _init():
        m_acc[...] = mp_ref[...]
        n_acc[0, 0] = _NORM_PERSISTENT

    q = q_ref[...]
    k = k_ref[...]

    row = jax.lax.broadcasted_iota(jnp.int32, (_CHUNK, _CHUNK), 0)
    col = jax.lax.broadcasted_iota(jnp.int32, (_CHUNK, _CHUNK), 1)
    causal = (col <= row)

    ss = jnp.sum(k * k, axis=1, keepdims=True)
    csum = jnp.dot(causal.astype(jnp.float32), ss,
                   preferred_element_type=jnp.float32)
    norms = n_acc[0, 0] + csum
    n_acc[0, 0] = n_acc[0, 0] + jnp.sum(ss)

    scores = jax.lax.dot_general(q, k, (((1,), (1,)), ((), ())),
                                 preferred_element_type=jnp.float32)
    scores = jnp.where(causal, scores, 0.0)

    m = m_acc[...]
    out = jax.lax.dot_general(q, m, (((1,), (1,)), ((), ())),
                              preferred_element_type=jnp.float32)
    out = out + jax.lax.dot_general(scores, k, (((1,), (0,)), ((), ())),
                                    preferred_element_type=jnp.float32)
    out_ref[...] = out / jnp.maximum(norms, 1e-8)

    m_acc[...] = m + jax.lax.dot_general(k, k, (((0,), (0,)), ((), ())),
                                         preferred_element_type=jnp.float32)


def kernel(queries, keys, m_persistent):
    t_len, dim = queries.shape
    n_chunks = t_len // _CHUNK
    return pl.pallas_call(
        _qkproj_kernel,
        out_shape=jax.ShapeDtypeStruct((t_len, dim), jnp.float32),
        grid=(n_chunks,),
        in_specs=[
            pl.BlockSpec((_CHUNK, dim), lambda i: (i, 0)),
            pl.BlockSpec((_CHUNK, dim), lambda i: (i, 0)),
            pl.BlockSpec((dim, dim), lambda i: (0, 0)),
        ],
        out_specs=pl.BlockSpec((_CHUNK, dim), lambda i: (i, 0)),
        scratch_shapes=[
            pltpu.VMEM((dim, dim), jnp.float32),
            pltpu.SMEM((1, 1), jnp.float32),
        ],
        compiler_params=pltpu.CompilerParams(
            dimension_semantics=("arbitrary",),
        ),
        name="qkprojection",
    )(queries, keys, m_persistent)
